# baseline (speedup 1.0000x reference)
import functools
import jax
import jax.numpy as jnp
from jax import lax
from jax.experimental import pallas as pl
from jax.experimental.pallas import tpu as pltpu
from jax.experimental.pallas import tpu_sc as plsc

N = 4096
D = 1024
E = 64
DF = 512
C = 128
TB = 512
NB = N // TB
TRASH = E * C
EPG = 4
EC_PAD = E * C + EPG * C

NC, NS = 2, 16
NW = NC * NS
TPW = N // NW


def _silu(z):
    return z / (1.0 + jnp.exp(-z))


def _router_shared_body(x_ref, wr_ref, b_ref, swg_ref, swu_ref, swd_ref,
                        shared_ref, eidx_ref, gate16_ref, d_ref, counts_ref):
    i = pl.program_id(0)

    @pl.when(i == 0)
    def _():
        counts_ref[...] = jnp.zeros((1, E), jnp.float32)

    xb = x_ref[...]
    logits = jax.lax.dot_general(
        xb, wr_ref[...], (((1,), (0,)), ((), ())),
        preferred_element_type=jnp.float32) + b_ref[...]

    lmax = jnp.max(logits, axis=1, keepdims=True)
    s = jnp.sum(jnp.exp(logits - lmax), axis=1, keepdims=True)
    pmax = 1.0 / s
    gate = pmax / (pmax + 1e-9)

    col = lax.broadcasted_iota(jnp.int32, (TB, E), 1).astype(jnp.float32)
    is_max = logits == lmax
    e_f = jnp.min(jnp.where(is_max, col, jnp.float32(E)), axis=1,
                  keepdims=True)
    onehot = (col == e_f).astype(jnp.float32)

    r_iota = lax.broadcasted_iota(jnp.int32, (TB, TB), 0)
    c_iota = lax.broadcasted_iota(jnp.int32, (TB, TB), 1)
    ltri = (r_iota >= c_iota).astype(jnp.float32)
    inc = jax.lax.dot_general(ltri, onehot, (((1,), (0,)), ((), ())),
                              preferred_element_type=jnp.float32)
    rank = jnp.sum(inc * onehot, axis=1, keepdims=True)
    prev = jnp.sum(counts_ref[...] * onehot, axis=1, keepdims=True)
    slot = prev + rank - 1.0
    counts_ref[...] = counts_ref[...] + jnp.sum(onehot, axis=0, keepdims=True)

    keep = slot < jnp.float32(C)
    d = jnp.where(keep, e_f * jnp.float32(C) + slot, jnp.float32(TRASH))
    d_ref[...] = d.astype(jnp.int32)
    eidx_ref[...] = e_f.astype(jnp.int32)
    gate16_ref[...] = jnp.broadcast_to(gate, (TB, 128))

    hg = jax.lax.dot_general(xb, swg_ref[...], (((1,), (0,)), ((), ())),
                             preferred_element_type=jnp.float32)
    hu = jax.lax.dot_general(xb, swu_ref[...], (((1,), (0,)), ((), ())),
                             preferred_element_type=jnp.float32)
    h = _silu(hg) * hu
    shared_ref[...] = jax.lax.dot_general(
        h, swd_ref[...], (((1,), (0,)), ((), ())),
        preferred_element_type=jnp.float32)


def _run_router_shared(xf, W_router, bias, sw_gate, sw_up, sw_down):
    return pl.pallas_call(
        _router_shared_body,
        grid=(NB,),
        in_specs=[
            pl.BlockSpec((TB, D), lambda i: (i, 0)),
            pl.BlockSpec((D, E), lambda i: (0, 0)),
            pl.BlockSpec((1, E), lambda i: (0, 0)),
            pl.BlockSpec((D, DF), lambda i: (0, 0)),
            pl.BlockSpec((D, DF), lambda i: (0, 0)),
            pl.BlockSpec((DF, D), lambda i: (0, 0)),
        ],
        out_specs=[
            pl.BlockSpec((TB, D), lambda i: (i, 0)),
            pl.BlockSpec((TB, 1), lambda i: (i, 0)),
            pl.BlockSpec((TB, 128), lambda i: (i, 0)),
            pl.BlockSpec((TB, 1), lambda i: (i, 0)),
        ],
        out_shape=[
            jax.ShapeDtypeStruct((N, D), jnp.float32),
            jax.ShapeDtypeStruct((N, 1), jnp.int32),
            jax.ShapeDtypeStruct((N, 128), jnp.float32),
            jax.ShapeDtypeStruct((N, 1), jnp.int32),
        ],
        scratch_shapes=[pltpu.VMEM((1, E), jnp.float32)],
    )(xf, W_router, bias, sw_gate, sw_up, sw_down)


def _expert_ffn_body(xin_ref, wg_ref, wu_ref, wd_ref, gs_ref, y_ref):
    e = pl.program_id(0)
    ntr = e < (E // EPG)
    for k in range(EPG):
        xb = xin_ref[pl.ds(k * C, C), :]
        xb = jnp.where(ntr, xb, jnp.zeros_like(xb))
        wg = wg_ref[k]
        wu = wu_ref[k]
        wd = wd_ref[k]
        g = jax.lax.dot_general(xb, wg, (((1,), (1,)), ((), ())),
                                preferred_element_type=jnp.float32)
        u = jax.lax.dot_general(xb, wu, (((1,), (1,)), ((), ())),
                                preferred_element_type=jnp.float32)
        h = _silu(g) * u
        y = jax.lax.dot_general(h, wd, (((1,), (1,)), ((), ())),
                                preferred_element_type=jnp.float32)
        y = y * gs_ref[pl.ds(k * C, C), :1]
        y_ref[pl.ds(k * C, C), :] = jnp.where(ntr, y, jnp.zeros_like(y))


def _run_expert_ffn(expert_in, w_gate, w_up, w_down, gate_slots):
    G = E // EPG
    wix = lambda e: (jnp.minimum(e, G - 1), 0, 0)
    return pl.pallas_call(
        _expert_ffn_body,
        grid=(G + 1,),
        in_specs=[
            pl.BlockSpec((EPG * C, D), lambda e: (e, 0)),
            pl.BlockSpec((EPG, DF, D), wix),
            pl.BlockSpec((EPG, DF, D), wix),
            pl.BlockSpec((EPG, D, DF), wix),
            pl.BlockSpec((EPG * C, 128), lambda e: (e, 0)),
        ],
        out_specs=pl.BlockSpec((EPG * C, D), lambda e: (e, 0)),
        out_shape=jax.ShapeDtypeStruct((EC_PAD, D), jnp.float32),
    )(expert_in, w_gate, w_up, w_down, gate_slots)


@functools.cache
def _sc_mesh():
    return plsc.VectorSubcoreMesh(core_axis_name="c", subcore_axis_name="s",
                                  num_cores=NC, num_subcores=NS)


_BCH = 64


def _scatter_body(x_hbm, g16_hbm, d_hbm, out_hbm, gs_hbm,
                  idx_v, rows_v, g_v, sem, gsem):
    wid = lax.axis_index("s") * NC + lax.axis_index("c")
    base = wid * TPW
    for cch in range(TPW // _BCH):
        off = base + cch * _BCH
        pltpu.sync_copy(d_hbm.at[pl.ds(off, _BCH)], idx_v)
        pltpu.sync_copy(x_hbm.at[pl.ds(off, _BCH)], rows_v)
        pltpu.sync_copy(g16_hbm.at[pl.ds(off, _BCH)], g_v)
        row_cp = pltpu.async_copy(rows_v, out_hbm.at[idx_v], sem)
        g_cp = pltpu.async_copy(g_v, gs_hbm.at[idx_v], gsem)
        row_cp.wait()
        g_cp.wait()


@functools.cache
def _run_scatter():
    return pl.kernel(
        _scatter_body,
        out_type=(jax.ShapeDtypeStruct((EC_PAD, D), jnp.float32),
                  jax.ShapeDtypeStruct((EC_PAD, 128), jnp.float32)),
        mesh=_sc_mesh(),
        scratch_types=[
            pltpu.VMEM((_BCH,), jnp.int32),
            pltpu.VMEM((_BCH, D), jnp.float32),
            pltpu.VMEM((_BCH, 128), jnp.float32),
            pltpu.SemaphoreType.DMA,
            pltpu.SemaphoreType.DMA,
        ],
    )


_DCH = 64


def _combine_body(y_hbm, d_hbm, out_hbm, idx_v, y_v, sem):
    wid = lax.axis_index("s") * NC + lax.axis_index("c")
    base = wid * TPW
    for cch in range(TPW // _DCH):
        off = base + cch * _DCH
        pltpu.sync_copy(d_hbm.at[pl.ds(off, _DCH)], idx_v)
        pltpu.async_copy(y_hbm.at[idx_v], y_v, sem).wait()
        pltpu.sync_copy(y_v, out_hbm.at[pl.ds(off, _DCH)])


@functools.cache
def _run_combine():
    return pl.kernel(
        _combine_body,
        out_type=jax.ShapeDtypeStruct((N, D), jnp.float32),
        mesh=_sc_mesh(),
        scratch_types=[
            pltpu.VMEM((_DCH,), jnp.int32),
            pltpu.VMEM((_DCH, D), jnp.float32),
            pltpu.SemaphoreType.DMA,
        ],
    )


def _final_add_body(sh_ref, ys_ref, out_ref):
    out_ref[...] = sh_ref[...] + ys_ref[...]


def _run_final_add(shared, ysorted):
    return pl.pallas_call(
        _final_add_body,
        grid=(NB,),
        in_specs=[pl.BlockSpec((TB, D), lambda i: (i, 0)),
                  pl.BlockSpec((TB, D), lambda i: (i, 0))],
        out_specs=pl.BlockSpec((TB, D), lambda i: (i, 0)),
        out_shape=jax.ShapeDtypeStruct((N, D), jnp.float32),
    )(shared, ysorted)


def kernel(x, W_router, bias, sw_gate, sw_up, sw_down, w_gate, w_up, w_down):
    Bb, Tt, Dm = x.shape
    xf = x.reshape(N, D)
    shared, eidx, gate16, d = _run_router_shared(
        xf, W_router, bias[:1], sw_gate, sw_up, sw_down)
    d1 = d.reshape(N)
    expert_in, gate_slots = _run_scatter()(xf, gate16, d1)
    y = _run_expert_ffn(expert_in, w_gate, w_up, w_down, gate_slots)
    ysorted = _run_combine()(y, d1)
    out = _run_final_add(shared, ysorted)
    aux_loss = jnp.zeros((), jnp.float32)
    return (out.reshape(Bb, Tt, Dm), aux_loss, eidx.reshape(Bb, Tt, 1))


# speedup vs baseline: 1.0146x; 1.0146x over previous
import functools
import jax
import jax.numpy as jnp
from jax import lax
from jax.experimental import pallas as pl
from jax.experimental.pallas import tpu as pltpu
from jax.experimental.pallas import tpu_sc as plsc

N = 4096
D = 1024
E = 64
DF = 512
C = 128
TB = 512
NB = N // TB
TRASH = E * C
EPG = 2
EC_PAD = E * C + EPG * C

NC, NS = 2, 16
NW = NC * NS
TPW = N // NW


def _silu(z):
    return z / (1.0 + jnp.exp(-z))


def _router_shared_body(x_ref, wr_ref, b_ref, swg_ref, swu_ref, swd_ref,
                        shared_ref, eidx_ref, gate16_ref, d_ref, counts_ref):
    i = pl.program_id(0)

    @pl.when(i == 0)
    def _():
        counts_ref[...] = jnp.zeros((1, E), jnp.float32)

    xb = x_ref[...]
    logits = jax.lax.dot_general(
        xb, wr_ref[...], (((1,), (0,)), ((), ())),
        preferred_element_type=jnp.float32) + b_ref[...]

    lmax = jnp.max(logits, axis=1, keepdims=True)
    s = jnp.sum(jnp.exp(logits - lmax), axis=1, keepdims=True)
    pmax = 1.0 / s
    gate = pmax / (pmax + 1e-9)

    col = lax.broadcasted_iota(jnp.int32, (TB, E), 1).astype(jnp.float32)
    is_max = logits == lmax
    e_f = jnp.min(jnp.where(is_max, col, jnp.float32(E)), axis=1,
                  keepdims=True)
    onehot = (col == e_f).astype(jnp.float32)

    r_iota = lax.broadcasted_iota(jnp.int32, (TB, TB), 0)
    c_iota = lax.broadcasted_iota(jnp.int32, (TB, TB), 1)
    ltri = (r_iota >= c_iota).astype(jnp.float32)
    inc = jax.lax.dot_general(ltri, onehot, (((1,), (0,)), ((), ())),
                              preferred_element_type=jnp.float32)
    rank = jnp.sum(inc * onehot, axis=1, keepdims=True)
    prev = jnp.sum(counts_ref[...] * onehot, axis=1, keepdims=True)
    slot = prev + rank - 1.0
    counts_ref[...] = counts_ref[...] + jnp.sum(onehot, axis=0, keepdims=True)

    keep = slot < jnp.float32(C)
    d = jnp.where(keep, e_f * jnp.float32(C) + slot, jnp.float32(TRASH))
    d_ref[...] = d.astype(jnp.int32)
    eidx_ref[...] = e_f.astype(jnp.int32)
    gate16_ref[...] = jnp.broadcast_to(gate, (TB, 128))

    hg = jax.lax.dot_general(xb, swg_ref[...], (((1,), (0,)), ((), ())),
                             preferred_element_type=jnp.float32)
    hu = jax.lax.dot_general(xb, swu_ref[...], (((1,), (0,)), ((), ())),
                             preferred_element_type=jnp.float32)
    h = _silu(hg) * hu
    shared_ref[...] = jax.lax.dot_general(
        h, swd_ref[...], (((1,), (0,)), ((), ())),
        preferred_element_type=jnp.float32)


def _run_router_shared(xf, W_router, bias, sw_gate, sw_up, sw_down):
    return pl.pallas_call(
        _router_shared_body,
        grid=(NB,),
        in_specs=[
            pl.BlockSpec((TB, D), lambda i: (i, 0)),
            pl.BlockSpec((D, E), lambda i: (0, 0)),
            pl.BlockSpec((1, E), lambda i: (0, 0)),
            pl.BlockSpec((D, DF), lambda i: (0, 0)),
            pl.BlockSpec((D, DF), lambda i: (0, 0)),
            pl.BlockSpec((DF, D), lambda i: (0, 0)),
        ],
        out_specs=[
            pl.BlockSpec((TB, D), lambda i: (i, 0)),
            pl.BlockSpec((TB, 1), lambda i: (i, 0)),
            pl.BlockSpec((TB, 128), lambda i: (i, 0)),
            pl.BlockSpec((TB, 1), lambda i: (i, 0)),
        ],
        out_shape=[
            jax.ShapeDtypeStruct((N, D), jnp.float32),
            jax.ShapeDtypeStruct((N, 1), jnp.int32),
            jax.ShapeDtypeStruct((N, 128), jnp.float32),
            jax.ShapeDtypeStruct((N, 1), jnp.int32),
        ],
        scratch_shapes=[pltpu.VMEM((1, E), jnp.float32)],
    )(xf, W_router, bias, sw_gate, sw_up, sw_down)


def _expert_ffn_body(xin_ref, wg_ref, wu_ref, wd_ref, gs_ref, y_ref):
    e = pl.program_id(0)
    ntr = e < (E // EPG)
    for k in range(EPG):
        xb = xin_ref[pl.ds(k * C, C), :]
        xb = jnp.where(ntr, xb, jnp.zeros_like(xb))
        wg = wg_ref[k]
        wu = wu_ref[k]
        wd = wd_ref[k]
        g = jax.lax.dot_general(xb, wg, (((1,), (1,)), ((), ())),
                                preferred_element_type=jnp.float32)
        u = jax.lax.dot_general(xb, wu, (((1,), (1,)), ((), ())),
                                preferred_element_type=jnp.float32)
        h = _silu(g) * u
        y = jax.lax.dot_general(h, wd, (((1,), (1,)), ((), ())),
                                preferred_element_type=jnp.float32)
        y = y * gs_ref[pl.ds(k * C, C), :1]
        y_ref[pl.ds(k * C, C), :] = jnp.where(ntr, y, jnp.zeros_like(y))


def _run_expert_ffn(expert_in, w_gate, w_up, w_down, gate_slots):
    G = E // EPG
    wix = lambda e: (jnp.minimum(e, G - 1), 0, 0)
    return pl.pallas_call(
        _expert_ffn_body,
        grid=(G + 1,),
        in_specs=[
            pl.BlockSpec((EPG * C, D), lambda e: (e, 0)),
            pl.BlockSpec((EPG, DF, D), wix),
            pl.BlockSpec((EPG, DF, D), wix),
            pl.BlockSpec((EPG, D, DF), wix),
            pl.BlockSpec((EPG * C, 128), lambda e: (e, 0)),
        ],
        out_specs=pl.BlockSpec((EPG * C, D), lambda e: (e, 0)),
        out_shape=jax.ShapeDtypeStruct((EC_PAD, D), jnp.float32),
    )(expert_in, w_gate, w_up, w_down, gate_slots)


@functools.cache
def _sc_mesh():
    return plsc.VectorSubcoreMesh(core_axis_name="c", subcore_axis_name="s",
                                  num_cores=NC, num_subcores=NS)


_BCH = 64


def _scatter_body(x_hbm, g16_hbm, d_hbm, out_hbm, gs_hbm,
                  idx_v, rows_v, g_v, sem, gsem):
    wid = lax.axis_index("s") * NC + lax.axis_index("c")
    base = wid * TPW
    for cch in range(TPW // _BCH):
        off = base + cch * _BCH
        pltpu.sync_copy(d_hbm.at[pl.ds(off, _BCH)], idx_v)
        pltpu.sync_copy(x_hbm.at[pl.ds(off, _BCH)], rows_v)
        pltpu.sync_copy(g16_hbm.at[pl.ds(off, _BCH)], g_v)
        row_cp = pltpu.async_copy(rows_v, out_hbm.at[idx_v], sem)
        g_cp = pltpu.async_copy(g_v, gs_hbm.at[idx_v], gsem)
        row_cp.wait()
        g_cp.wait()


@functools.cache
def _run_scatter():
    return pl.kernel(
        _scatter_body,
        out_type=(jax.ShapeDtypeStruct((EC_PAD, D), jnp.float32),
                  jax.ShapeDtypeStruct((EC_PAD, 128), jnp.float32)),
        mesh=_sc_mesh(),
        scratch_types=[
            pltpu.VMEM((_BCH,), jnp.int32),
            pltpu.VMEM((_BCH, D), jnp.float32),
            pltpu.VMEM((_BCH, 128), jnp.float32),
            pltpu.SemaphoreType.DMA,
            pltpu.SemaphoreType.DMA,
        ],
    )


_DCH = 32


def _combine_body(y_hbm, sh_hbm, d_hbm, out_hbm, idx_v, y_v, s_v, sem, ssem):
    wid = lax.axis_index("s") * NC + lax.axis_index("c")
    base = wid * TPW
    for cch in range(TPW // _DCH):
        off = base + cch * _DCH
        pltpu.sync_copy(d_hbm.at[pl.ds(off, _DCH)], idx_v)
        s_cp = pltpu.async_copy(sh_hbm.at[pl.ds(off, _DCH)], s_v, ssem)
        pltpu.async_copy(y_hbm.at[idx_v], y_v, sem).wait()
        s_cp.wait()

        @plsc.parallel_loop(0, _DCH * (D // 16), unroll=8)
        def _(i):
            r = lax.shift_right_logical(i, 6)
            c = pl.multiple_of(
                lax.shift_left(jnp.bitwise_and(i, (D // 16) - 1), 4), 16)
            s_v[r, pl.ds(c, 16)] = s_v[r, pl.ds(c, 16)] + y_v[r, pl.ds(c, 16)]

        pltpu.sync_copy(s_v, out_hbm.at[pl.ds(off, _DCH)])


@functools.cache
def _run_combine():
    return pl.kernel(
        _combine_body,
        out_type=jax.ShapeDtypeStruct((N, D), jnp.float32),
        mesh=_sc_mesh(),
        scratch_types=[
            pltpu.VMEM((_DCH,), jnp.int32),
            pltpu.VMEM((_DCH, D), jnp.float32),
            pltpu.VMEM((_DCH, D), jnp.float32),
            pltpu.SemaphoreType.DMA,
            pltpu.SemaphoreType.DMA,
        ],
    )


def kernel(x, W_router, bias, sw_gate, sw_up, sw_down, w_gate, w_up, w_down):
    Bb, Tt, Dm = x.shape
    xf = x.reshape(N, D)
    shared, eidx, gate16, d = _run_router_shared(
        xf, W_router, bias[:1], sw_gate, sw_up, sw_down)
    d1 = d.reshape(N)
    expert_in, gate_slots = _run_scatter()(xf, gate16, d1)
    y = _run_expert_ffn(expert_in, w_gate, w_up, w_down, gate_slots)
    out = _run_combine()(y, shared, d1)
    aux_loss = jnp.zeros((), jnp.float32)
    return (out.reshape(Bb, Tt, Dm), aux_loss, eidx.reshape(Bb, Tt, 1))


# speedup vs baseline: 1.0307x; 1.0158x over previous
import functools
import jax
import jax.numpy as jnp
from jax import lax
from jax.experimental import pallas as pl
from jax.experimental.pallas import tpu as pltpu
from jax.experimental.pallas import tpu_sc as plsc

N = 4096
D = 1024
E = 64
DF = 512
C = 128
TB = 512
NB = N // TB
TRASH = E * C
EPG = 2
EC_PAD = E * C + EPG * C

NC, NS = 2, 16
NW = NC * NS
TPW = N // NW


def _silu(z):
    return z / (1.0 + jnp.exp(-z))


def _router_body(x_ref, wr_ref, b_ref,
                 eidx_ref, gate16_ref, d_ref, counts_ref):
    i = pl.program_id(0)

    @pl.when(i == 0)
    def _():
        counts_ref[...] = jnp.zeros((1, E), jnp.float32)

    xb = x_ref[...]
    logits = jax.lax.dot_general(
        xb, wr_ref[...], (((1,), (0,)), ((), ())),
        preferred_element_type=jnp.float32) + b_ref[...]

    lmax = jnp.max(logits, axis=1, keepdims=True)
    s = jnp.sum(jnp.exp(logits - lmax), axis=1, keepdims=True)
    pmax = 1.0 / s
    gate = pmax / (pmax + 1e-9)

    col = lax.broadcasted_iota(jnp.int32, (TB, E), 1).astype(jnp.float32)
    is_max = logits == lmax
    e_f = jnp.min(jnp.where(is_max, col, jnp.float32(E)), axis=1,
                  keepdims=True)
    onehot = (col == e_f).astype(jnp.float32)

    r_iota = lax.broadcasted_iota(jnp.int32, (TB, TB), 0)
    c_iota = lax.broadcasted_iota(jnp.int32, (TB, TB), 1)
    ltri = (r_iota >= c_iota).astype(jnp.float32)
    inc = jax.lax.dot_general(ltri, onehot, (((1,), (0,)), ((), ())),
                              preferred_element_type=jnp.float32)
    rank = jnp.sum(inc * onehot, axis=1, keepdims=True)
    prev = jnp.sum(counts_ref[...] * onehot, axis=1, keepdims=True)
    slot = prev + rank - 1.0
    counts_ref[...] = counts_ref[...] + jnp.sum(onehot, axis=0, keepdims=True)

    keep = slot < jnp.float32(C)
    d = jnp.where(keep, e_f * jnp.float32(C) + slot, jnp.float32(TRASH))
    d_ref[...] = d.astype(jnp.int32)
    eidx_ref[...] = e_f.astype(jnp.int32)
    gate16_ref[...] = jnp.broadcast_to(gate, (TB, 128))


def _run_router(xf, W_router, bias):
    return pl.pallas_call(
        _router_body,
        grid=(NB,),
        in_specs=[
            pl.BlockSpec((TB, D), lambda i: (i, 0)),
            pl.BlockSpec((D, E), lambda i: (0, 0)),
            pl.BlockSpec((1, E), lambda i: (0, 0)),
        ],
        out_specs=[
            pl.BlockSpec((TB, 1), lambda i: (i, 0)),
            pl.BlockSpec((TB, 128), lambda i: (i, 0)),
            pl.BlockSpec((TB, 1), lambda i: (i, 0)),
        ],
        out_shape=[
            jax.ShapeDtypeStruct((N, 1), jnp.int32),
            jax.ShapeDtypeStruct((N, 128), jnp.float32),
            jax.ShapeDtypeStruct((N, 1), jnp.int32),
        ],
        scratch_shapes=[pltpu.VMEM((1, E), jnp.float32)],
    )(xf, W_router, bias)


def _shared_body(x_ref, swg_ref, swu_ref, swd_ref, shared_ref):
    xb = x_ref[...]
    hg = jax.lax.dot_general(xb, swg_ref[...], (((1,), (0,)), ((), ())),
                             preferred_element_type=jnp.float32)
    hu = jax.lax.dot_general(xb, swu_ref[...], (((1,), (0,)), ((), ())),
                             preferred_element_type=jnp.float32)
    h = _silu(hg) * hu
    shared_ref[...] = jax.lax.dot_general(
        h, swd_ref[...], (((1,), (0,)), ((), ())),
        preferred_element_type=jnp.float32)


def _run_shared(xf, sw_gate, sw_up, sw_down):
    return pl.pallas_call(
        _shared_body,
        grid=(NB,),
        in_specs=[
            pl.BlockSpec((TB, D), lambda i: (i, 0)),
            pl.BlockSpec((D, DF), lambda i: (0, 0)),
            pl.BlockSpec((D, DF), lambda i: (0, 0)),
            pl.BlockSpec((DF, D), lambda i: (0, 0)),
        ],
        out_specs=pl.BlockSpec((TB, D), lambda i: (i, 0)),
        out_shape=jax.ShapeDtypeStruct((N, D), jnp.float32),
    )(xf, sw_gate, sw_up, sw_down)


def _expert_ffn_body(xin_ref, wg_ref, wu_ref, wd_ref, gs_ref, y_ref):
    e = pl.program_id(0)
    ntr = e < (E // EPG)
    for k in range(EPG):
        xb = xin_ref[pl.ds(k * C, C), :]
        xb = jnp.where(ntr, xb, jnp.zeros_like(xb))
        wg = wg_ref[k]
        wu = wu_ref[k]
        wd = wd_ref[k]
        g = jax.lax.dot_general(xb, wg, (((1,), (1,)), ((), ())),
                                preferred_element_type=jnp.float32)
        u = jax.lax.dot_general(xb, wu, (((1,), (1,)), ((), ())),
                                preferred_element_type=jnp.float32)
        h = _silu(g) * u
        y = jax.lax.dot_general(h, wd, (((1,), (1,)), ((), ())),
                                preferred_element_type=jnp.float32)
        y = y * gs_ref[pl.ds(k * C, C), :1]
        y_ref[pl.ds(k * C, C), :] = jnp.where(ntr, y, jnp.zeros_like(y))


def _run_expert_ffn(expert_in, w_gate, w_up, w_down, gate_slots):
    G = E // EPG
    wix = lambda e: (jnp.minimum(e, G - 1), 0, 0)
    return pl.pallas_call(
        _expert_ffn_body,
        grid=(G + 1,),
        in_specs=[
            pl.BlockSpec((EPG * C, D), lambda e: (e, 0)),
            pl.BlockSpec((EPG, DF, D), wix),
            pl.BlockSpec((EPG, DF, D), wix),
            pl.BlockSpec((EPG, D, DF), wix),
            pl.BlockSpec((EPG * C, 128), lambda e: (e, 0)),
        ],
        out_specs=pl.BlockSpec((EPG * C, D), lambda e: (e, 0)),
        out_shape=jax.ShapeDtypeStruct((EC_PAD, D), jnp.float32),
    )(expert_in, w_gate, w_up, w_down, gate_slots)


@functools.cache
def _sc_mesh():
    return plsc.VectorSubcoreMesh(core_axis_name="c", subcore_axis_name="s",
                                  num_cores=NC, num_subcores=NS)


_BCH = 64


def _scatter_body(x_hbm, g16_hbm, d_hbm, out_hbm, gs_hbm,
                  idx_v, rows_v, g_v, sem, gsem):
    wid = lax.axis_index("s") * NC + lax.axis_index("c")
    base = wid * TPW
    for cch in range(TPW // _BCH):
        off = base + cch * _BCH
        pltpu.sync_copy(d_hbm.at[pl.ds(off, _BCH)], idx_v)
        pltpu.sync_copy(x_hbm.at[pl.ds(off, _BCH)], rows_v)
        pltpu.sync_copy(g16_hbm.at[pl.ds(off, _BCH)], g_v)
        row_cp = pltpu.async_copy(rows_v, out_hbm.at[idx_v], sem)
        g_cp = pltpu.async_copy(g_v, gs_hbm.at[idx_v], gsem)
        row_cp.wait()
        g_cp.wait()


@functools.cache
def _run_scatter():
    return pl.kernel(
        _scatter_body,
        out_type=(jax.ShapeDtypeStruct((EC_PAD, D), jnp.float32),
                  jax.ShapeDtypeStruct((EC_PAD, 128), jnp.float32)),
        mesh=_sc_mesh(),
        scratch_types=[
            pltpu.VMEM((_BCH,), jnp.int32),
            pltpu.VMEM((_BCH, D), jnp.float32),
            pltpu.VMEM((_BCH, 128), jnp.float32),
            pltpu.SemaphoreType.DMA,
            pltpu.SemaphoreType.DMA,
        ],
    )


_DCH = 32


def _combine_body(y_hbm, sh_hbm, d_hbm, out_hbm, idx_v, y_v, s_v, sem, ssem):
    wid = lax.axis_index("s") * NC + lax.axis_index("c")
    base = wid * TPW
    for cch in range(TPW // _DCH):
        off = base + cch * _DCH
        pltpu.sync_copy(d_hbm.at[pl.ds(off, _DCH)], idx_v)
        s_cp = pltpu.async_copy(sh_hbm.at[pl.ds(off, _DCH)], s_v, ssem)
        pltpu.async_copy(y_hbm.at[idx_v], y_v, sem).wait()
        s_cp.wait()

        @plsc.parallel_loop(0, _DCH * (D // 16), unroll=8)
        def _(i):
            r = lax.shift_right_logical(i, 6)
            c = pl.multiple_of(
                lax.shift_left(jnp.bitwise_and(i, (D // 16) - 1), 4), 16)
            s_v[r, pl.ds(c, 16)] = s_v[r, pl.ds(c, 16)] + y_v[r, pl.ds(c, 16)]

        pltpu.sync_copy(s_v, out_hbm.at[pl.ds(off, _DCH)])


@functools.cache
def _run_combine():
    return pl.kernel(
        _combine_body,
        out_type=jax.ShapeDtypeStruct((N, D), jnp.float32),
        mesh=_sc_mesh(),
        scratch_types=[
            pltpu.VMEM((_DCH,), jnp.int32),
            pltpu.VMEM((_DCH, D), jnp.float32),
            pltpu.VMEM((_DCH, D), jnp.float32),
            pltpu.SemaphoreType.DMA,
            pltpu.SemaphoreType.DMA,
        ],
    )


def kernel(x, W_router, bias, sw_gate, sw_up, sw_down, w_gate, w_up, w_down):
    Bb, Tt, Dm = x.shape
    xf = x.reshape(N, D)
    eidx, gate16, d = _run_router(xf, W_router, bias[:1])
    shared = _run_shared(xf, sw_gate, sw_up, sw_down)
    d1 = d.reshape(N)
    expert_in, gate_slots = _run_scatter()(xf, gate16, d1)
    y = _run_expert_ffn(expert_in, w_gate, w_up, w_down, gate_slots)
    out = _run_combine()(y, shared, d1)
    aux_loss = jnp.zeros((), jnp.float32)
    return (out.reshape(Bb, Tt, Dm), aux_loss, eidx.reshape(Bb, Tt, 1))


# speedup vs baseline: 1.0311x; 1.0004x over previous
import functools
import jax
import jax.numpy as jnp
from jax import lax
from jax.experimental import pallas as pl
from jax.experimental.pallas import tpu as pltpu
from jax.experimental.pallas import tpu_sc as plsc

N = 4096
D = 1024
E = 64
DF = 512
C = 128
TB = 512
NB = N // TB
TRASH = E * C
EPG = 2
EC_PAD = E * C + EPG * C

NC, NS = 2, 16
NW = NC * NS
TPW = N // NW


def _silu(z):
    return z / (1.0 + jnp.exp(-z))


def _router_body(x_ref, wr_ref, b_ref,
                 eidx_ref, gate16_ref, d_ref, counts_ref):
    i = pl.program_id(0)

    @pl.when(i == 0)
    def _():
        counts_ref[...] = jnp.zeros((1, E), jnp.float32)

    xb = x_ref[...]
    logits = jax.lax.dot_general(
        xb, wr_ref[...], (((1,), (0,)), ((), ())),
        preferred_element_type=jnp.float32) + b_ref[...]

    lmax = jnp.max(logits, axis=1, keepdims=True)
    s = jnp.sum(jnp.exp(logits - lmax), axis=1, keepdims=True)
    pmax = 1.0 / s
    gate = pmax / (pmax + 1e-9)

    col = lax.broadcasted_iota(jnp.int32, (TB, E), 1).astype(jnp.float32)
    is_max = logits == lmax
    e_f = jnp.min(jnp.where(is_max, col, jnp.float32(E)), axis=1,
                  keepdims=True)
    onehot = (col == e_f).astype(jnp.float32)

    r_iota = lax.broadcasted_iota(jnp.int32, (TB, TB), 0)
    c_iota = lax.broadcasted_iota(jnp.int32, (TB, TB), 1)
    ltri = (r_iota >= c_iota).astype(jnp.float32)
    inc = jax.lax.dot_general(ltri, onehot, (((1,), (0,)), ((), ())),
                              preferred_element_type=jnp.float32)
    rank = jnp.sum(inc * onehot, axis=1, keepdims=True)
    prev = jnp.sum(counts_ref[...] * onehot, axis=1, keepdims=True)
    slot = prev + rank - 1.0
    counts_ref[...] = counts_ref[...] + jnp.sum(onehot, axis=0, keepdims=True)

    keep = slot < jnp.float32(C)
    d = jnp.where(keep, e_f * jnp.float32(C) + slot, jnp.float32(TRASH))
    d_ref[...] = d.astype(jnp.int32)
    eidx_ref[...] = e_f.astype(jnp.int32)
    gate16_ref[...] = jnp.broadcast_to(gate, (TB, 128))


def _run_router(xf, W_router, bias):
    return pl.pallas_call(
        _router_body,
        grid=(NB,),
        in_specs=[
            pl.BlockSpec((TB, D), lambda i: (i, 0)),
            pl.BlockSpec((D, E), lambda i: (0, 0)),
            pl.BlockSpec((1, E), lambda i: (0, 0)),
        ],
        out_specs=[
            pl.BlockSpec((TB, 1), lambda i: (i, 0)),
            pl.BlockSpec((TB, 128), lambda i: (i, 0)),
            pl.BlockSpec((TB, 1), lambda i: (i, 0)),
        ],
        out_shape=[
            jax.ShapeDtypeStruct((N, 1), jnp.int32),
            jax.ShapeDtypeStruct((N, 128), jnp.float32),
            jax.ShapeDtypeStruct((N, 1), jnp.int32),
        ],
        scratch_shapes=[pltpu.VMEM((1, E), jnp.float32)],
    )(xf, W_router, bias)


def _shared_body(x_ref, swg_ref, swu_ref, swd_ref, shared_ref):
    xb = x_ref[...]
    hg = jax.lax.dot_general(xb, swg_ref[...], (((1,), (0,)), ((), ())),
                             preferred_element_type=jnp.float32)
    hu = jax.lax.dot_general(xb, swu_ref[...], (((1,), (0,)), ((), ())),
                             preferred_element_type=jnp.float32)
    h = _silu(hg) * hu
    shared_ref[...] = jax.lax.dot_general(
        h, swd_ref[...], (((1,), (0,)), ((), ())),
        preferred_element_type=jnp.float32)


def _run_shared(xf, sw_gate, sw_up, sw_down):
    return pl.pallas_call(
        _shared_body,
        grid=(NB,),
        in_specs=[
            pl.BlockSpec((TB, D), lambda i: (i, 0)),
            pl.BlockSpec((D, DF), lambda i: (0, 0)),
            pl.BlockSpec((D, DF), lambda i: (0, 0)),
            pl.BlockSpec((DF, D), lambda i: (0, 0)),
        ],
        out_specs=pl.BlockSpec((TB, D), lambda i: (i, 0)),
        out_shape=jax.ShapeDtypeStruct((N, D), jnp.float32),
    )(xf, sw_gate, sw_up, sw_down)


def _expert_ffn_body(xin_ref, wg_ref, wu_ref, wd_ref, gs_ref, y_ref):
    e = pl.program_id(0)
    ntr = e < (E // EPG)
    for k in range(EPG):
        xb = xin_ref[pl.ds(k * C, C), :]
        xb = jnp.where(ntr, xb, jnp.zeros_like(xb))
        wg = wg_ref[k]
        wu = wu_ref[k]
        wd = wd_ref[k]
        g = jax.lax.dot_general(xb, wg, (((1,), (1,)), ((), ())),
                                preferred_element_type=jnp.float32)
        u = jax.lax.dot_general(xb, wu, (((1,), (1,)), ((), ())),
                                preferred_element_type=jnp.float32)
        h = _silu(g) * u
        y = jax.lax.dot_general(h, wd, (((1,), (1,)), ((), ())),
                                preferred_element_type=jnp.float32)
        y = y * gs_ref[pl.ds(k * C, C), :1]
        y_ref[pl.ds(k * C, C), :] = jnp.where(ntr, y, jnp.zeros_like(y))


def _run_expert_ffn(expert_in, w_gate, w_up, w_down, gate_slots):
    G = E // EPG
    wix = lambda e: (jnp.minimum(e, G - 1), 0, 0)
    return pl.pallas_call(
        _expert_ffn_body,
        grid=(G + 1,),
        in_specs=[
            pl.BlockSpec((EPG * C, D), lambda e: (e, 0)),
            pl.BlockSpec((EPG, DF, D), wix),
            pl.BlockSpec((EPG, DF, D), wix),
            pl.BlockSpec((EPG, D, DF), wix),
            pl.BlockSpec((EPG * C, 128), lambda e: (e, 0)),
        ],
        out_specs=pl.BlockSpec((EPG * C, D), lambda e: (e, 0)),
        out_shape=jax.ShapeDtypeStruct((EC_PAD, D), jnp.float32),
    )(expert_in, w_gate, w_up, w_down, gate_slots)


@functools.cache
def _sc_mesh():
    return plsc.VectorSubcoreMesh(core_axis_name="c", subcore_axis_name="s",
                                  num_cores=NC, num_subcores=NS)


_BCH = 64


def _scatter_body(x_hbm, g16_hbm, d_hbm, out_hbm, gs_hbm,
                  idx_v, rows_v, g_v, sem, gsem):
    wid = lax.axis_index("s") * NC + lax.axis_index("c")
    base = wid * TPW
    for cch in range(TPW // _BCH):
        off = base + cch * _BCH
        pltpu.sync_copy(d_hbm.at[pl.ds(off, _BCH)], idx_v)
        pltpu.sync_copy(x_hbm.at[pl.ds(off, _BCH)], rows_v)
        pltpu.sync_copy(g16_hbm.at[pl.ds(off, _BCH)], g_v)
        row_cp = pltpu.async_copy(rows_v, out_hbm.at[idx_v], sem)
        g_cp = pltpu.async_copy(g_v, gs_hbm.at[idx_v], gsem)
        row_cp.wait()
        g_cp.wait()


@functools.cache
def _run_scatter():
    return pl.kernel(
        _scatter_body,
        out_type=(jax.ShapeDtypeStruct((EC_PAD, D), jnp.float32),
                  jax.ShapeDtypeStruct((EC_PAD, 128), jnp.float32)),
        mesh=_sc_mesh(),
        scratch_types=[
            pltpu.VMEM((_BCH,), jnp.int32),
            pltpu.VMEM((_BCH, D), jnp.float32),
            pltpu.VMEM((_BCH, 128), jnp.float32),
            pltpu.SemaphoreType.DMA,
            pltpu.SemaphoreType.DMA,
        ],
    )


_DCH = 32


def _combine_body(y_hbm, sh_hbm, d_hbm, out_hbm, idx_v, y_v, s_v, sem, ssem):
    wid = lax.axis_index("s") * NC + lax.axis_index("c")
    base = wid * TPW
    for cch in range(TPW // _DCH):
        off = base + cch * _DCH
        pltpu.sync_copy(d_hbm.at[pl.ds(off, _DCH)], idx_v)
        s_cp = pltpu.async_copy(sh_hbm.at[pl.ds(off, _DCH)], s_v, ssem)
        pltpu.async_copy(y_hbm.at[idx_v], y_v, sem).wait()
        s_cp.wait()

        @plsc.parallel_loop(0, _DCH * (D // 16), unroll=16)
        def _(i):
            r = lax.shift_right_logical(i, 6)
            c = pl.multiple_of(
                lax.shift_left(jnp.bitwise_and(i, (D // 16) - 1), 4), 16)
            s_v[r, pl.ds(c, 16)] = s_v[r, pl.ds(c, 16)] + y_v[r, pl.ds(c, 16)]

        pltpu.sync_copy(s_v, out_hbm.at[pl.ds(off, _DCH)])


@functools.cache
def _run_combine():
    return pl.kernel(
        _combine_body,
        out_type=jax.ShapeDtypeStruct((N, D), jnp.float32),
        mesh=_sc_mesh(),
        scratch_types=[
            pltpu.VMEM((_DCH,), jnp.int32),
            pltpu.VMEM((_DCH, D), jnp.float32),
            pltpu.VMEM((_DCH, D), jnp.float32),
            pltpu.SemaphoreType.DMA,
            pltpu.SemaphoreType.DMA,
        ],
    )


def kernel(x, W_router, bias, sw_gate, sw_up, sw_down, w_gate, w_up, w_down):
    Bb, Tt, Dm = x.shape
    xf = x.reshape(N, D)
    eidx, gate16, d = _run_router(xf, W_router, bias[:1])
    shared = _run_shared(xf, sw_gate, sw_up, sw_down)
    d1 = d.reshape(N)
    expert_in, gate_slots = _run_scatter()(xf, gate16, d1)
    y = _run_expert_ffn(expert_in, w_gate, w_up, w_down, gate_slots)
    out = _run_combine()(y, shared, d1)
    aux_loss = jnp.zeros((), jnp.float32)
    return (out.reshape(Bb, Tt, Dm), aux_loss, eidx.reshape(Bb, Tt, 1))
